# baseline (speedup 1.0000x reference)
import jax
import jax.numpy as jnp
from jax.experimental import pallas as pl
from jax.experimental.pallas import tpu as pltpu

_B, _Wdim, _F = 16, 256, 64
_H, _D = 2, 256
_ALPHA = 0.2


def _gat_batch_kernel(x_ref, wt_ref, bb_ref, a04_ref, a06_ref, o_ref):
    xb = x_ref[0]
    p = jnp.dot(wt_ref[...], xb, preferred_element_type=jnp.float32)
    p = p + bb_ref[...]

    def head(h):
        s = p[h * _D:(h + 1) * _D]
        t = p[(2 + h) * _D:(3 + h) * _D]
        z = s[:, :, None] + t[:, None, :]
        e = jnp.sum(jnp.abs(z) * a04_ref[h][:, :, None], axis=0)
        slin = jnp.sum(s * a06_ref[h], axis=0)
        e = e + slin[:, None]
        m = jnp.max(e, axis=0, keepdims=True)
        ex = jnp.exp(e - m)
        a = ex / jnp.sum(ex, axis=0, keepdims=True)
        return jnp.dot(s, a, preferred_element_type=jnp.float32)

    o_ref[0] = 0.5 * (head(0) + head(1))


def kernel(x, W_src, b_src, W_dst, b_dst, attn):
    wt = jnp.concatenate([W_src.T, W_dst.T], axis=0)
    bb = jnp.concatenate([b_src, b_dst])[:, None]
    a04 = jnp.broadcast_to((0.4 * attn)[:, :, None], (_H, _D, _F))
    a06 = jnp.broadcast_to((0.6 * attn)[:, :, None], (_H, _D, _F))

    grid = (_B,)
    out = pl.pallas_call(
        _gat_batch_kernel,
        grid=grid,
        in_specs=[
            pl.BlockSpec((1, _Wdim, _F), lambda b: (b, 0, 0)),
            pl.BlockSpec((4 * _D, _Wdim), lambda b: (0, 0)),
            pl.BlockSpec((4 * _D, 1), lambda b: (0, 0)),
            pl.BlockSpec((_H, _D, _F), lambda b: (0, 0, 0)),
            pl.BlockSpec((_H, _D, _F), lambda b: (0, 0, 0)),
        ],
        out_specs=pl.BlockSpec((1, _D, _F), lambda b: (b, 0, 0)),
        out_shape=jax.ShapeDtypeStruct((_B, _D, _F), jnp.float32),
        compiler_params=pltpu.CompilerParams(
            dimension_semantics=("parallel",),
        ),
    )(x, wt, bb, a04, a06)
    return out


# speedup vs baseline: 3.0778x; 3.0778x over previous
import jax
import jax.numpy as jnp
from jax.experimental import pallas as pl
from jax.experimental.pallas import tpu as pltpu

_B, _Wdim, _F = 16, 256, 64
_H, _D = 2, 256
_ALPHA = 0.2
_NCOLS = 3 * _H * _D + 128


def _gat_batch_kernel(xt_ref, wt_ref, bb_ref, sgw_ref, o_ref, pa_ref, e_ref):
    xbt = xt_ref[0]
    pa_ref[...] = (jnp.dot(xbt, wt_ref[...], preferred_element_type=jnp.float32)
                   + bb_ref[...])

    outs = []
    for h in range(_H):
        sth = pa_ref[:, h * _D:(h + 1) * _D]
        sgc = sgw_ref[:, h:h + 1]
        for j in range(_F):
            ttrow = pa_ref[j, 512 + h * _D:512 + (h + 1) * _D][None, :]
            w = jnp.abs(sth + ttrow)
            e_ref[:, h * _F + j:h * _F + j + 1] = jnp.dot(
                w, sgc, preferred_element_type=jnp.float32)
        slin = pa_ref[:, 1536 + h:1537 + h]
        e = e_ref[:, h * _F:(h + 1) * _F] + slin
        m = jnp.max(e, axis=0, keepdims=True)
        ex = jnp.exp(e - m)
        a = ex / jnp.sum(ex, axis=0, keepdims=True)
        srawT = pa_ref[:, 1024 + h * _D:1024 + (h + 1) * _D]
        outs.append(jax.lax.dot_general(
            srawT, a, (((0,), (0,)), ((), ())),
            preferred_element_type=jnp.float32))

    o_ref[0] = 0.5 * (outs[0] + outs[1])


def kernel(x, W_src, b_src, W_dst, b_dst, attn):
    af = attn.reshape(_H * _D)
    sc = 0.4 * jnp.abs(af)
    wlin = jnp.stack([
        W_src[:, h * _D:(h + 1) * _D] @ (0.6 * attn[h]) for h in range(_H)
    ], axis=1)
    blin = jnp.stack([
        (0.6 * attn[h]) @ b_src[h * _D:(h + 1) * _D] for h in range(_H)
    ])
    wt = jnp.concatenate([
        W_src * sc[None, :], W_dst * sc[None, :], W_src, wlin,
        jnp.zeros((_Wdim, 126), jnp.float32),
    ], axis=1)
    bb = jnp.concatenate([
        b_src * sc, b_dst * sc, b_src, blin, jnp.zeros((126,), jnp.float32),
    ])[None, :]
    sgw = jnp.sign(attn).T
    xt = jnp.transpose(x, (0, 2, 1))

    grid = (_B,)
    out = pl.pallas_call(
        _gat_batch_kernel,
        grid=grid,
        in_specs=[
            pl.BlockSpec((1, _F, _Wdim), lambda b: (b, 0, 0)),
            pl.BlockSpec((_Wdim, _NCOLS), lambda b: (0, 0)),
            pl.BlockSpec((1, _NCOLS), lambda b: (0, 0)),
            pl.BlockSpec((_D, _H), lambda b: (0, 0)),
        ],
        out_specs=pl.BlockSpec((1, _D, _F), lambda b: (b, 0, 0)),
        out_shape=jax.ShapeDtypeStruct((_B, _D, _F), jnp.float32),
        scratch_shapes=[
            pltpu.VMEM((_F, _NCOLS), jnp.float32),
            pltpu.VMEM((_F, _H * _F), jnp.float32),
        ],
        compiler_params=pltpu.CompilerParams(
            dimension_semantics=("parallel",),
        ),
    )(xt, wt, bb, sgw)
    return out


# speedup vs baseline: 3.3119x; 1.0761x over previous
import jax
import jax.numpy as jnp
from jax.experimental import pallas as pl
from jax.experimental.pallas import tpu as pltpu

_B, _Wdim, _F = 16, 256, 64
_H, _D = 2, 256
_ALPHA = 0.2
_NCOLS = 3 * _H * _D + 128


def _gat_batch_kernel(xt_ref, wt_ref, bb_ref, sgw_ref, o_ref, pa_ref, pb_ref,
                      e_ref):
    xbt = xt_ref[0]
    pa_ref[...] = (jnp.dot(xbt, wt_ref[...], preferred_element_type=jnp.float32)
                   + bb_ref[...])
    pb_ref[...] = pa_ref[:, 0:2 * _H * _D].astype(jnp.bfloat16)

    outs = []
    for h in range(_H):
        sth = pb_ref[:, h * _D:(h + 1) * _D]
        sgc = sgw_ref[:, h:h + 1]
        for j in range(_F):
            ttrow = pb_ref[j, 512 + h * _D:512 + (h + 1) * _D][None, :]
            w = jnp.abs(sth + ttrow)
            e_ref[h * _F + j:h * _F + j + 1, :] = jax.lax.dot_general(
                sgc, w, (((0,), (1,)), ((), ())),
                preferred_element_type=jnp.float32)
        slin = pa_ref[:, 1536 + h:1537 + h]
        e_t = e_ref[h * _F:(h + 1) * _F, :] + jnp.transpose(slin)
        m = jnp.max(e_t, axis=1, keepdims=True)
        ex = jnp.exp(e_t - m)
        a_t = ex / jnp.sum(ex, axis=1, keepdims=True)
        srawT = pa_ref[:, 1024 + h * _D:1024 + (h + 1) * _D]
        outs.append(jax.lax.dot_general(
            srawT.astype(jnp.bfloat16), a_t.astype(jnp.bfloat16),
            (((0,), (1,)), ((), ())),
            preferred_element_type=jnp.float32))

    o_ref[0] = 0.5 * (outs[0] + outs[1])


def kernel(x, W_src, b_src, W_dst, b_dst, attn):
    af = attn.reshape(_H * _D)
    sc = 0.4 * jnp.abs(af)
    wlin = jnp.stack([
        W_src[:, h * _D:(h + 1) * _D] @ (0.6 * attn[h]) for h in range(_H)
    ], axis=1)
    blin = jnp.stack([
        (0.6 * attn[h]) @ b_src[h * _D:(h + 1) * _D] for h in range(_H)
    ])
    wt = jnp.concatenate([
        W_src * sc[None, :], W_dst * sc[None, :], W_src, wlin,
        jnp.zeros((_Wdim, 126), jnp.float32),
    ], axis=1)
    bb = jnp.concatenate([
        b_src * sc, b_dst * sc, b_src, blin, jnp.zeros((126,), jnp.float32),
    ])[None, :]
    sgw = jnp.sign(attn).T.astype(jnp.bfloat16)
    wt = wt.astype(jnp.bfloat16)
    xt = jnp.transpose(x, (0, 2, 1)).astype(jnp.bfloat16)

    grid = (_B,)
    out = pl.pallas_call(
        _gat_batch_kernel,
        grid=grid,
        in_specs=[
            pl.BlockSpec((1, _F, _Wdim), lambda b: (b, 0, 0)),
            pl.BlockSpec((_Wdim, _NCOLS), lambda b: (0, 0)),
            pl.BlockSpec((1, _NCOLS), lambda b: (0, 0)),
            pl.BlockSpec((_D, _H), lambda b: (0, 0)),
        ],
        out_specs=pl.BlockSpec((1, _D, _F), lambda b: (b, 0, 0)),
        out_shape=jax.ShapeDtypeStruct((_B, _D, _F), jnp.float32),
        scratch_shapes=[
            pltpu.VMEM((_F, _NCOLS), jnp.float32),
            pltpu.VMEM((_F, 2 * _H * _D), jnp.bfloat16),
            pltpu.VMEM((_H * _F, _F), jnp.float32),
        ],
        compiler_params=pltpu.CompilerParams(
            dimension_semantics=("parallel",),
        ),
    )(xt, wt, bb, sgw)
    return out


# speedup vs baseline: 3.5972x; 1.0861x over previous
import jax
import jax.numpy as jnp
from jax.experimental import pallas as pl
from jax.experimental.pallas import tpu as pltpu

_B, _Wdim, _F = 16, 256, 64
_H, _D = 2, 256
_ALPHA = 0.2
_NB = 4
_NCOLS = 3 * _H * _D + 128


def _gat_batch_kernel(xt_ref, wt_ref, bb_ref, sgw_ref, o_ref, pa_ref, pb_ref,
                      e_ref):
    for bb in range(_NB):
        xbt = xt_ref[bb]
        pa_ref[bb] = (jnp.dot(xbt, wt_ref[...],
                              preferred_element_type=jnp.float32)
                      + bb_ref[...])
        pb_ref[bb] = pa_ref[bb, :, 0:2 * _H * _D].astype(jnp.bfloat16)

    for bb in range(_NB):
        outs = []
        for h in range(_H):
            sth = pb_ref[bb, :, h * _D:(h + 1) * _D]
            sgc = sgw_ref[:, h:h + 1]
            for j in range(_F):
                ttrow = pb_ref[bb, j, 512 + h * _D:512 + (h + 1) * _D][None, :]
                w = jnp.abs(sth + ttrow)
                e_ref[bb, h * _F + j:h * _F + j + 1, :] = jax.lax.dot_general(
                    sgc, w, (((0,), (1,)), ((), ())),
                    preferred_element_type=jnp.float32)
            slin = pa_ref[bb, :, 1536 + h:1537 + h]
            e_t = e_ref[bb, h * _F:(h + 1) * _F, :] + jnp.transpose(slin)
            m = jnp.max(e_t, axis=1, keepdims=True)
            ex = jnp.exp(e_t - m)
            a_t = ex / jnp.sum(ex, axis=1, keepdims=True)
            srawT = pa_ref[bb, :, 1024 + h * _D:1024 + (h + 1) * _D]
            outs.append(jax.lax.dot_general(
                srawT.astype(jnp.bfloat16), a_t.astype(jnp.bfloat16),
                (((0,), (1,)), ((), ())),
                preferred_element_type=jnp.float32))

        o_ref[bb] = 0.5 * (outs[0] + outs[1])


def kernel(x, W_src, b_src, W_dst, b_dst, attn):
    af = attn.reshape(_H * _D)
    sc = 0.4 * jnp.abs(af)
    wlin = jnp.stack([
        W_src[:, h * _D:(h + 1) * _D] @ (0.6 * attn[h]) for h in range(_H)
    ], axis=1)
    blin = jnp.stack([
        (0.6 * attn[h]) @ b_src[h * _D:(h + 1) * _D] for h in range(_H)
    ])
    wt = jnp.concatenate([
        W_src * sc[None, :], W_dst * sc[None, :], W_src, wlin,
        jnp.zeros((_Wdim, 126), jnp.float32),
    ], axis=1)
    bb = jnp.concatenate([
        b_src * sc, b_dst * sc, b_src, blin, jnp.zeros((126,), jnp.float32),
    ])[None, :]
    sgw = jnp.sign(attn).T.astype(jnp.bfloat16)
    wt = wt.astype(jnp.bfloat16)
    xt = jnp.transpose(x, (0, 2, 1)).astype(jnp.bfloat16)

    grid = (_B // _NB,)
    out = pl.pallas_call(
        _gat_batch_kernel,
        grid=grid,
        in_specs=[
            pl.BlockSpec((_NB, _F, _Wdim), lambda b: (b, 0, 0)),
            pl.BlockSpec((_Wdim, _NCOLS), lambda b: (0, 0)),
            pl.BlockSpec((1, _NCOLS), lambda b: (0, 0)),
            pl.BlockSpec((_D, _H), lambda b: (0, 0)),
        ],
        out_specs=pl.BlockSpec((_NB, _D, _F), lambda b: (b, 0, 0)),
        out_shape=jax.ShapeDtypeStruct((_B, _D, _F), jnp.float32),
        scratch_shapes=[
            pltpu.VMEM((_NB, _F, _NCOLS), jnp.float32),
            pltpu.VMEM((_NB, _F, 2 * _H * _D), jnp.bfloat16),
            pltpu.VMEM((_NB, _H * _F, _F), jnp.float32),
        ],
        compiler_params=pltpu.CompilerParams(
            dimension_semantics=("parallel",),
        ),
    )(xt, wt, bb, sgw)
    return out


# speedup vs baseline: 3.6221x; 1.0069x over previous
import jax
import jax.numpy as jnp
from jax.experimental import pallas as pl
from jax.experimental.pallas import tpu as pltpu

_B, _Wdim, _F = 16, 256, 64
_H, _D = 2, 256
_ALPHA = 0.2
_NB = 8
_NCOLS = 3 * _H * _D + 128


def _gat_batch_kernel(xt_ref, wt_ref, bb_ref, sgw_ref, o_ref, pa_ref, pb_ref,
                      e_ref):
    for bb in range(_NB):
        xbt = xt_ref[bb]
        pa_ref[bb] = (jnp.dot(xbt, wt_ref[...],
                              preferred_element_type=jnp.float32)
                      + bb_ref[...])
        pb_ref[bb] = pa_ref[bb, :, 0:2 * _H * _D].astype(jnp.bfloat16)

    for bb in range(_NB):
        outs = []
        for h in range(_H):
            sth = pb_ref[bb, :, h * _D:(h + 1) * _D]
            sgc = sgw_ref[:, h:h + 1]
            for j in range(_F):
                ttrow = pb_ref[bb, j, 512 + h * _D:512 + (h + 1) * _D][None, :]
                w = jnp.abs(sth + ttrow)
                e_ref[bb, h * _F + j:h * _F + j + 1, :] = jax.lax.dot_general(
                    sgc, w, (((0,), (1,)), ((), ())),
                    preferred_element_type=jnp.float32)
            slin = pa_ref[bb, :, 1536 + h:1537 + h]
            e_t = e_ref[bb, h * _F:(h + 1) * _F, :] + jnp.transpose(slin)
            m = jnp.max(e_t, axis=1, keepdims=True)
            ex = jnp.exp(e_t - m)
            a_t = ex / jnp.sum(ex, axis=1, keepdims=True)
            srawT = pa_ref[bb, :, 1024 + h * _D:1024 + (h + 1) * _D]
            outs.append(jax.lax.dot_general(
                srawT.astype(jnp.bfloat16), a_t.astype(jnp.bfloat16),
                (((0,), (1,)), ((), ())),
                preferred_element_type=jnp.float32))

        o_ref[bb] = 0.5 * (outs[0] + outs[1])


def kernel(x, W_src, b_src, W_dst, b_dst, attn):
    af = attn.reshape(_H * _D)
    sc = 0.4 * jnp.abs(af)
    wlin = jnp.stack([
        W_src[:, h * _D:(h + 1) * _D] @ (0.6 * attn[h]) for h in range(_H)
    ], axis=1)
    blin = jnp.stack([
        (0.6 * attn[h]) @ b_src[h * _D:(h + 1) * _D] for h in range(_H)
    ])
    wt = jnp.concatenate([
        W_src * sc[None, :], W_dst * sc[None, :], W_src, wlin,
        jnp.zeros((_Wdim, 126), jnp.float32),
    ], axis=1)
    bb = jnp.concatenate([
        b_src * sc, b_dst * sc, b_src, blin, jnp.zeros((126,), jnp.float32),
    ])[None, :]
    sgw = jnp.sign(attn).T.astype(jnp.bfloat16)
    wt = wt.astype(jnp.bfloat16)
    xt = jnp.transpose(x, (0, 2, 1)).astype(jnp.bfloat16)

    grid = (_B // _NB,)
    out = pl.pallas_call(
        _gat_batch_kernel,
        grid=grid,
        in_specs=[
            pl.BlockSpec((_NB, _F, _Wdim), lambda b: (b, 0, 0)),
            pl.BlockSpec((_Wdim, _NCOLS), lambda b: (0, 0)),
            pl.BlockSpec((1, _NCOLS), lambda b: (0, 0)),
            pl.BlockSpec((_D, _H), lambda b: (0, 0)),
        ],
        out_specs=pl.BlockSpec((_NB, _D, _F), lambda b: (b, 0, 0)),
        out_shape=jax.ShapeDtypeStruct((_B, _D, _F), jnp.float32),
        scratch_shapes=[
            pltpu.VMEM((_NB, _F, _NCOLS), jnp.float32),
            pltpu.VMEM((_NB, _F, 2 * _H * _D), jnp.bfloat16),
            pltpu.VMEM((_NB, _H * _F, _F), jnp.float32),
        ],
        compiler_params=pltpu.CompilerParams(
            dimension_semantics=("parallel",),
        ),
    )(xt, wt, bb, sgw)
    return out


# speedup vs baseline: 4.0259x; 1.1115x over previous
import jax
import jax.numpy as jnp
from jax.experimental import pallas as pl
from jax.experimental.pallas import tpu as pltpu

_B, _Wdim, _F = 16, 256, 64
_H, _D = 2, 256
_ALPHA = 0.2
_NB = 8
_NCOLS = 3 * _H * _D + 128


def _gat_batch_kernel(xt_ref, wt_ref, bb_ref, sgw_ref, o_ref, pa_ref, pb_ref,
                      e_ref):
    for bb in range(_NB):
        xbt = xt_ref[bb]
        pa_ref[bb] = (jnp.dot(xbt, wt_ref[...],
                              preferred_element_type=jnp.float32)
                      + bb_ref[...])
        pb_ref[bb] = pa_ref[bb, :, 0:2 * _H * _D].astype(jnp.bfloat16)

    def emit_logits(bb, h):
        sth = pb_ref[bb, :, h * _D:(h + 1) * _D]
        sgc = sgw_ref[:, h:h + 1]
        for j in range(_F):
            ttrow = pb_ref[bb, j, 512 + h * _D:512 + (h + 1) * _D][None, :]
            w = jnp.abs(sth + ttrow)
            e_ref[bb, h * _F + j:h * _F + j + 1, :] = jax.lax.dot_general(
                sgc, w, (((0,), (1,)), ((), ())),
                preferred_element_type=jnp.float32)

    def emit_tail(bb, h):
        slin = pa_ref[bb, :, 1536 + h:1537 + h]
        e_t = e_ref[bb, h * _F:(h + 1) * _F, :] + jnp.transpose(slin)
        m = jnp.max(e_t, axis=1, keepdims=True)
        ex = jnp.exp(e_t - m)
        a_t = ex / jnp.sum(ex, axis=1, keepdims=True)
        srawT = pa_ref[bb, :, 1024 + h * _D:1024 + (h + 1) * _D]
        return jax.lax.dot_general(
            srawT.astype(jnp.bfloat16), a_t.astype(jnp.bfloat16),
            (((0,), (1,)), ((), ())),
            preferred_element_type=jnp.float32)

    units = [(bb, h) for bb in range(_NB) for h in range(_H)]
    outs = {}
    for k, (bb, h) in enumerate(units):
        emit_logits(bb, h)
        if k > 0:
            pbb, ph = units[k - 1]
            outs[(pbb, ph)] = emit_tail(pbb, ph)
            if ph == _H - 1:
                o_ref[pbb] = 0.5 * (outs[(pbb, 0)] + outs[(pbb, 1)])
    lbb, lh = units[-1]
    outs[(lbb, lh)] = emit_tail(lbb, lh)
    o_ref[lbb] = 0.5 * (outs[(lbb, 0)] + outs[(lbb, 1)])


def kernel(x, W_src, b_src, W_dst, b_dst, attn):
    af = attn.reshape(_H * _D)
    sc = 0.4 * jnp.abs(af)
    wlin = jnp.stack([
        W_src[:, h * _D:(h + 1) * _D] @ (0.6 * attn[h]) for h in range(_H)
    ], axis=1)
    blin = jnp.stack([
        (0.6 * attn[h]) @ b_src[h * _D:(h + 1) * _D] for h in range(_H)
    ])
    wt = jnp.concatenate([
        W_src * sc[None, :], W_dst * sc[None, :], W_src, wlin,
        jnp.zeros((_Wdim, 126), jnp.float32),
    ], axis=1)
    bb = jnp.concatenate([
        b_src * sc, b_dst * sc, b_src, blin, jnp.zeros((126,), jnp.float32),
    ])[None, :]
    sgw = jnp.sign(attn).T.astype(jnp.bfloat16)
    wt = wt.astype(jnp.bfloat16)
    xt = jnp.transpose(x, (0, 2, 1)).astype(jnp.bfloat16)

    grid = (_B // _NB,)
    out = pl.pallas_call(
        _gat_batch_kernel,
        grid=grid,
        in_specs=[
            pl.BlockSpec((_NB, _F, _Wdim), lambda b: (b, 0, 0)),
            pl.BlockSpec((_Wdim, _NCOLS), lambda b: (0, 0)),
            pl.BlockSpec((1, _NCOLS), lambda b: (0, 0)),
            pl.BlockSpec((_D, _H), lambda b: (0, 0)),
        ],
        out_specs=pl.BlockSpec((_NB, _D, _F), lambda b: (b, 0, 0)),
        out_shape=jax.ShapeDtypeStruct((_B, _D, _F), jnp.float32),
        scratch_shapes=[
            pltpu.VMEM((_NB, _F, _NCOLS), jnp.float32),
            pltpu.VMEM((_NB, _F, 2 * _H * _D), jnp.bfloat16),
            pltpu.VMEM((_NB, _H * _F, _F), jnp.float32),
        ],
        compiler_params=pltpu.CompilerParams(
            dimension_semantics=("parallel",),
        ),
    )(xt, wt, bb, sgw)
    return out
